# baseline (speedup 1.0000x reference)
import jax
import jax.numpy as jnp
from jax.experimental import pallas as pl
from jax.experimental.pallas import tpu as pltpu

QUEUE_SIZE = 262144
DIM = 128
BATCH = 4096
SUB = 8192
NSTEP = QUEUE_SIZE // SUB
NSUB = SUB // BATCH
KD_R = BATCH // NSTEP
NBUF = 4


def _body(p_ref, kpad_ref, q_ref, out_ref, kd_ref, ibuf, obuf, isems, osems):
    i = pl.program_id(0)
    s = jax.lax.rem(i, NBUF)
    p = jnp.clip(p_ref[0], 0, QUEUE_SIZE - BATCH)
    row_start = i * SUB

    def _in_copy(step, slot):
        return pltpu.make_async_copy(
            q_ref.at[pl.ds(step * SUB, SUB), :],
            ibuf.at[slot],
            isems.at[slot],
        )

    def _out_copy(step, slot):
        return pltpu.make_async_copy(
            obuf.at[slot],
            out_ref.at[:, pl.ds(step * SUB, SUB)],
            osems.at[slot],
        )

    @pl.when(i == 0)
    def _():
        for a in range(NBUF):
            _in_copy(a, a).start()

    @pl.when(i >= NBUF)
    def _():
        _out_copy(i - NBUF, s).wait()

    _in_copy(i, s).wait()

    overlap = jnp.logical_and(row_start + SUB > p, row_start < p + BATCH)

    @pl.when(overlap)
    def _():
        for j in range(NSUB):
            sub_start = row_start + j * BATCH
            start = jnp.clip(sub_start - p, -BATCH, BATCH) + BATCH
            kblk = kpad_ref[pl.ds(start, BATCH), :]
            rows = sub_start + jax.lax.broadcasted_iota(
                jnp.int32, (BATCH, 1), 0
            )
            mask = jnp.logical_and(rows >= p, rows < p + BATCH)
            qsub = ibuf[s, pl.ds(j * BATCH, BATCH), :]
            obuf[s, :, pl.ds(j * BATCH, BATCH)] = jnp.where(
                mask, kblk, qsub
            ).T

    @pl.when(jnp.logical_not(overlap))
    def _():
        obuf[s, ...] = ibuf[s, ...].T

    _out_copy(i, s).start()

    @pl.when(i + NBUF < NSTEP)
    def _():
        _in_copy(i + NBUF, s).start()

    kd_ref[...] = kpad_ref[pl.ds(BATCH + i * KD_R, KD_R), :]

    @pl.when(i == NSTEP - 1)
    def _():
        for b in range(NBUF):
            step = NSTEP - NBUF + b
            _out_copy(step, jax.lax.rem(jnp.int32(step), NBUF)).wait()


@jax.jit
def _fused(kpad, queue, ptr):
    grid_spec = pltpu.PrefetchScalarGridSpec(
        num_scalar_prefetch=1,
        grid=(NSTEP,),
        in_specs=[
            pl.BlockSpec((3 * BATCH, DIM), lambda i, p: (0, 0)),
            pl.BlockSpec(memory_space=pl.ANY),
        ],
        out_specs=[
            pl.BlockSpec(memory_space=pl.ANY),
            pl.BlockSpec((KD_R, DIM), lambda i, p: (i, 0)),
        ],
        scratch_shapes=[
            pltpu.VMEM((NBUF, SUB, DIM), jnp.float32),
            pltpu.VMEM((NBUF, DIM, SUB), jnp.float32),
            pltpu.SemaphoreType.DMA((NBUF,)),
            pltpu.SemaphoreType.DMA((NBUF,)),
        ],
    )
    return pl.pallas_call(
        _body,
        grid_spec=grid_spec,
        compiler_params=pltpu.CompilerParams(
            vmem_limit_bytes=128 * 1024 * 1024
        ),
        out_shape=[
            jax.ShapeDtypeStruct((DIM, QUEUE_SIZE), jnp.float32),
            jax.ShapeDtypeStruct((BATCH, DIM), jnp.float32),
        ],
    )(ptr, kpad, queue)


def kernel(k, queue, queue_ptr):
    k = jax.lax.stop_gradient(k)
    kpad = jnp.concatenate(
        [
            jnp.zeros((BATCH, DIM), jnp.float32),
            k,
            jnp.zeros((BATCH, DIM), jnp.float32),
        ]
    )
    ptr = jnp.atleast_1d(jnp.asarray(queue_ptr, jnp.int32))
    queue_t, kd = _fused(kpad, queue, ptr)
    return (kd, queue_t)


# speedup vs baseline: 1.0644x; 1.0644x over previous
import jax
import jax.numpy as jnp
from jax.experimental import pallas as pl
from jax.experimental.pallas import tpu as pltpu

QUEUE_SIZE = 262144
DIM = 128
BATCH = 4096
SUB = 16384
NSTEP = QUEUE_SIZE // SUB
NSUB = SUB // 1024
KD_R = BATCH // NSTEP
INBUF = 4
OUTBUF = 3
SUBC = 1024


def _body(p_ref, kpad_ref, q_ref, out_ref, kd_ref, ibuf, obuf, isems, osems):
    i = pl.program_id(0)
    si = jax.lax.rem(i, INBUF)
    so = jax.lax.rem(i, OUTBUF)
    p = jnp.clip(p_ref[0], 0, QUEUE_SIZE - BATCH)
    row_start = i * SUB

    def _in_copy(step, slot):
        return pltpu.make_async_copy(
            q_ref.at[pl.ds(step * SUB, SUB), :],
            ibuf.at[slot],
            isems.at[slot],
        )

    def _out_copy(step, slot):
        return pltpu.make_async_copy(
            obuf.at[slot],
            out_ref.at[:, pl.ds(step * SUB, SUB)],
            osems.at[slot],
        )

    @pl.when(i == 0)
    def _():
        for a in range(INBUF):
            _in_copy(a, a).start()

    @pl.when(i >= OUTBUF)
    def _():
        _out_copy(i - OUTBUF, so).wait()

    _in_copy(i, si).wait()

    overlap = jnp.logical_and(row_start + SUB > p, row_start < p + BATCH)

    @pl.when(overlap)
    def _():
        for j in range(NSUB):
            sub_start = row_start + j * SUBC
            start = jnp.clip(sub_start - p, -SUBC, BATCH) + SUBC
            kblk = kpad_ref[pl.ds(start, SUBC), :]
            rows = sub_start + jax.lax.broadcasted_iota(
                jnp.int32, (SUBC, 1), 0
            )
            mask = jnp.logical_and(rows >= p, rows < p + BATCH)
            qsub = ibuf[si, pl.ds(j * SUBC, SUBC), :]
            obuf[so, :, pl.ds(j * SUBC, SUBC)] = jnp.where(
                mask, kblk, qsub
            ).T

    @pl.when(jnp.logical_not(overlap))
    def _():
        obuf[so, ...] = ibuf[si, ...].T

    _out_copy(i, so).start()

    @pl.when(i + INBUF < NSTEP)
    def _():
        _in_copy(i + INBUF, si).start()

    kd_ref[...] = kpad_ref[pl.ds(SUBC + i * KD_R, KD_R), :]

    @pl.when(i == NSTEP - 1)
    def _():
        for b in range(OUTBUF):
            step = NSTEP - OUTBUF + b
            _out_copy(step, jax.lax.rem(jnp.int32(step), OUTBUF)).wait()


@jax.jit
def _fused(kpad, queue, ptr):
    grid_spec = pltpu.PrefetchScalarGridSpec(
        num_scalar_prefetch=1,
        grid=(NSTEP,),
        in_specs=[
            pl.BlockSpec((BATCH + 2 * SUBC, DIM), lambda i, p: (0, 0)),
            pl.BlockSpec(memory_space=pl.ANY),
        ],
        out_specs=[
            pl.BlockSpec(memory_space=pl.ANY),
            pl.BlockSpec((KD_R, DIM), lambda i, p: (i, 0)),
        ],
        scratch_shapes=[
            pltpu.VMEM((INBUF, SUB, DIM), jnp.float32),
            pltpu.VMEM((OUTBUF, DIM, SUB), jnp.float32),
            pltpu.SemaphoreType.DMA((INBUF,)),
            pltpu.SemaphoreType.DMA((OUTBUF,)),
        ],
    )
    return pl.pallas_call(
        _body,
        grid_spec=grid_spec,
        compiler_params=pltpu.CompilerParams(
            vmem_limit_bytes=128 * 1024 * 1024
        ),
        out_shape=[
            jax.ShapeDtypeStruct((DIM, QUEUE_SIZE), jnp.float32),
            jax.ShapeDtypeStruct((BATCH, DIM), jnp.float32),
        ],
    )(ptr, kpad, queue)


SUBC_HOST = SUBC


def kernel(k, queue, queue_ptr):
    k = jax.lax.stop_gradient(k)
    kpad = jnp.concatenate(
        [
            jnp.zeros((SUBC_HOST, DIM), jnp.float32),
            k,
            jnp.zeros((SUBC_HOST, DIM), jnp.float32),
        ]
    )
    ptr = jnp.atleast_1d(jnp.asarray(queue_ptr, jnp.int32))
    queue_t, kd = _fused(kpad, queue, ptr)
    return (kd, queue_t)
